# baseline (speedup 1.0000x reference)
import jax
import jax.numpy as jnp
from jax.experimental import pallas as pl

N = 10000
G = 64


def _copy_kernel(x_ref, o_ref):
    o_ref[...] = x_ref[...]


def _bn(x, gamma, beta):
    mu = jnp.mean(x, axis=0)
    var = jnp.var(x, axis=0)
    return (x - mu) / jnp.sqrt(var + 1e-5) * gamma + beta


def _edge_conv(x, src, dst, W1, b1, W2, b2):
    xi = x[dst]
    xj = x[src]
    h = jnp.concatenate([xi, xj - xi], axis=1)
    h = jax.nn.relu(h @ W1 + b1)
    h = jax.nn.relu(h @ W2 + b2)
    out = jax.ops.segment_max(h, dst, num_segments=x.shape[0])
    return jnp.where(jnp.isfinite(out), out, 0.0)


def kernel(x, edge_index, batch, W1a, b1a, W2a, b2a, g1, be1, W1b, b1b, W2b, b2b, g2, be2, Wp1, bp1, gp, bep, Wp2, bp2):
    src = edge_index[0]
    dst = edge_index[1]
    h = _edge_conv(x, src, dst, W1a, b1a, W2a, b2a)
    h = _bn(h, g1, be1)
    h = _edge_conv(h, src, dst, W1b, b1b, W2b, b2b)
    x_node = _bn(h, g2, be2)
    ones = jnp.ones((N,), jnp.float32)
    counts = jax.ops.segment_sum(ones, batch, num_segments=G)
    mean_pool = jax.ops.segment_sum(x_node, batch, num_segments=G) / jnp.clip(counts, 1.0)[:, None]
    max_pool = jax.ops.segment_max(x_node, batch, num_segments=G)
    max_pool = jnp.where(jnp.isfinite(max_pool), max_pool, 0.0)
    x_graph = jnp.concatenate([mean_pool, max_pool], axis=1)
    z = x_graph @ Wp1 + bp1
    z = _bn(z, gp, bep)
    z = jax.nn.relu(z)
    z_proj = z @ Wp2 + bp2
    z_proj = pl.pallas_call(
        _copy_kernel,
        out_shape=jax.ShapeDtypeStruct(z_proj.shape, z_proj.dtype),
    )(z_proj)
    return (z_proj, x_node, x_graph)


# speedup vs baseline: 1.3226x; 1.3226x over previous
import functools

import jax
import jax.numpy as jnp
from jax import lax
from jax.experimental import pallas as pl
from jax.experimental.pallas import tpu as pltpu
from jax.experimental.pallas import tpu_sc as plsc

N = 10000
E = 320000
G = 64
NSUB = 4
EBS = 1280
NBS = 2000
NEG = float("-inf")


def _edge_mlp_body(Din, xi_ref, xj_ref, w1t_ref, w1b_ref, b1_ref, w2_ref, b2_ref, o_ref):
    xi = xi_ref[...][:, :Din]
    xj = xj_ref[...][:, :Din]
    h = jnp.dot(xi, w1t_ref[...]) + jnp.dot(xj - xi, w1b_ref[...]) + b1_ref[...]
    h = jnp.maximum(h, 0.0)
    h = jnp.dot(h, w2_ref[...]) + b2_ref[...]
    o_ref[...] = lax.transpose(jnp.maximum(h, 0.0), (1, 0))


def _edge_mlp(xi, xj, W1, b1, W2, b2):
    Dpad = xi.shape[1]
    D = W1.shape[0] // 2
    H = W2.shape[1]
    nb = E // EBS
    return pl.pallas_call(
        functools.partial(_edge_mlp_body, D),
        grid=(nb,),
        in_specs=[
            pl.BlockSpec((EBS, Dpad), lambda i: (i, 0)),
            pl.BlockSpec((EBS, Dpad), lambda i: (i, 0)),
            pl.BlockSpec((D, H), lambda i: (0, 0)),
            pl.BlockSpec((D, H), lambda i: (0, 0)),
            pl.BlockSpec((1, H), lambda i: (0, 0)),
            pl.BlockSpec((H, H), lambda i: (0, 0)),
            pl.BlockSpec((1, H), lambda i: (0, 0)),
        ],
        out_specs=pl.BlockSpec((H, EBS), lambda i: (0, i)),
        out_shape=jax.ShapeDtypeStruct((H, E), jnp.float32),
    )(xi, xj, W1[:D], W1[D:], b1.reshape(1, H), W2, b2.reshape(1, H))


def _combine_stats_body(p_ref, maxed_ref, stats_ref):
    i = pl.program_id(0)
    m = p_ref[0]
    for q in range(1, NSUB):
        m = jnp.maximum(m, p_ref[q])
    m = jnp.where(jnp.isfinite(m), m, 0.0)
    maxed_ref[...] = m

    @pl.when(i == 0)
    def _():
        stats_ref[...] = jnp.zeros_like(stats_ref)

    s = jnp.sum(m, axis=0, keepdims=True)
    ss = jnp.sum(m * m, axis=0, keepdims=True)
    stats_ref[...] += jnp.concatenate([s, ss], axis=0)


def _combine_stats(partials):
    H = partials.shape[2]
    nb = N // NBS
    return pl.pallas_call(
        _combine_stats_body,
        grid=(nb,),
        in_specs=[pl.BlockSpec((NSUB, NBS, H), lambda i: (0, i, 0))],
        out_specs=[
            pl.BlockSpec((NBS, H), lambda i: (i, 0)),
            pl.BlockSpec((2, H), lambda i: (0, 0)),
        ],
        out_shape=[
            jax.ShapeDtypeStruct((N, H), jnp.float32),
            jax.ShapeDtypeStruct((2, H), jnp.float32),
        ],
    )(partials)


def _bn_apply_body(pad, x_ref, stats_ref, g_ref, be_ref, o_ref):
    s = stats_ref[0:1]
    ss = stats_ref[1:2]
    mu = s / float(N)
    var = ss / float(N) - mu * mu
    rstd = lax.rsqrt(var + 1e-5)
    y = (x_ref[...] - mu) * rstd * g_ref[...] + be_ref[...]
    if pad:
        y = jnp.concatenate([y, jnp.zeros_like(y)], axis=1)
    o_ref[...] = y


def _bn_apply(x, stats, gamma, beta, pad):
    H = x.shape[1]
    Ho = 2 * H if pad else H
    nb = N // NBS
    return pl.pallas_call(
        functools.partial(_bn_apply_body, pad),
        grid=(nb,),
        in_specs=[
            pl.BlockSpec((NBS, H), lambda i: (i, 0)),
            pl.BlockSpec((2, H), lambda i: (0, 0)),
            pl.BlockSpec((1, H), lambda i: (0, 0)),
            pl.BlockSpec((1, H), lambda i: (0, 0)),
        ],
        out_specs=pl.BlockSpec((NBS, Ho), lambda i: (i, 0)),
        out_shape=jax.ShapeDtypeStruct((N, Ho), jnp.float32),
    )(x, stats, gamma.reshape(1, H), beta.reshape(1, H))


def _pool_acc_body(xn_ref, b_ref, cnt_ref, sum_ref, max_ref):
    i = pl.program_id(0)

    @pl.when(i == 0)
    def _():
        cnt_ref[...] = jnp.zeros_like(cnt_ref)
        sum_ref[...] = jnp.zeros_like(sum_ref)
        max_ref[...] = jnp.full_like(max_ref, NEG)

    xn = xn_ref[...]
    b = b_ref[...]
    iota = lax.broadcasted_iota(jnp.int32, (NBS, G), 1)
    mf = (b == iota).astype(jnp.float32)
    cnt_ref[...] += jnp.sum(mf, axis=0, keepdims=True)
    sum_ref[...] += lax.dot_general(
        mf, xn, (((0,), (0,)), ((), ())), precision=lax.Precision.HIGHEST
    )

    def body(g, _):
        mask = b == g
        mg = jnp.max(jnp.where(mask, xn, NEG), axis=0, keepdims=True)
        max_ref[pl.ds(g, 1), :] = jnp.maximum(max_ref[pl.ds(g, 1), :], mg)
        return 0

    lax.fori_loop(0, G, body, 0)


def _pool_acc(x_node, batch_col):
    H = x_node.shape[1]
    nb = N // NBS
    return pl.pallas_call(
        _pool_acc_body,
        grid=(nb,),
        in_specs=[
            pl.BlockSpec((NBS, H), lambda i: (i, 0)),
            pl.BlockSpec((NBS, 1), lambda i: (i, 0)),
        ],
        out_specs=[
            pl.BlockSpec((1, G), lambda i: (0, 0)),
            pl.BlockSpec((G, H), lambda i: (0, 0)),
            pl.BlockSpec((G, H), lambda i: (0, 0)),
        ],
        out_shape=[
            jax.ShapeDtypeStruct((1, G), jnp.float32),
            jax.ShapeDtypeStruct((G, H), jnp.float32),
            jax.ShapeDtypeStruct((G, H), jnp.float32),
        ],
    )(x_node, batch_col)


def _final_body(cnt_ref, sum_ref, max_ref, wp1_ref, bp1_ref, gp_ref, bep_ref,
                wp2_ref, bp2_ref, zp_ref, xg_ref):
    cnt = jnp.maximum(cnt_ref[...], 1.0)
    mean_pool = sum_ref[...] / cnt.reshape(G, 1)
    mx = max_ref[...]
    max_pool = jnp.where(jnp.isfinite(mx), mx, 0.0)
    x_graph = jnp.concatenate([mean_pool, max_pool], axis=1)
    xg_ref[...] = x_graph
    z = jnp.dot(x_graph, wp1_ref[...]) + bp1_ref[...]
    mu = jnp.mean(z, axis=0, keepdims=True)
    var = jnp.mean((z - mu) * (z - mu), axis=0, keepdims=True)
    z = (z - mu) * lax.rsqrt(var + 1e-5) * gp_ref[...] + bep_ref[...]
    z = jnp.maximum(z, 0.0)
    zp_ref[...] = jnp.dot(z, wp2_ref[...]) + bp2_ref[...]


def _final(cnt, sums, maxs, Wp1, bp1, gp, bep, Wp2, bp2):
    H = sums.shape[1]
    P = Wp1.shape[1]
    return pl.pallas_call(
        _final_body,
        out_shape=[
            jax.ShapeDtypeStruct((G, P), jnp.float32),
            jax.ShapeDtypeStruct((G, 2 * H), jnp.float32),
        ],
    )(cnt, sums, maxs, Wp1, bp1.reshape(1, P), gp.reshape(1, P),
      bep.reshape(1, P), Wp2, bp2.reshape(1, P))


_NC = 2
_NS = 16
_NW = _NC * _NS
_GK = 80
_MESH = dict(core_axis_name="c", subcore_axis_name="s")


def _make_gather(Dh):
    per_w = E // _NW
    nch = per_w // _GK

    @functools.partial(
        pl.kernel,
        mesh=plsc.VectorSubcoreMesh(**_MESH),
        out_type=[
            jax.ShapeDtypeStruct((E, Dh), jnp.float32),
            jax.ShapeDtypeStruct((E, Dh), jnp.float32),
        ],
        scratch_types=[
            pltpu.VMEM((_GK,), jnp.int32),
            pltpu.VMEM((_GK, Dh), jnp.float32),
            pltpu.SemaphoreType.DMA,
        ],
    )
    def k(table_hbm, dst_hbm, src_hbm, xi_hbm, xj_hbm, idx_v, rows_v, sem):
        wid = lax.axis_index("s") * _NC + lax.axis_index("c")
        base = wid * per_w

        def body(i, _):
            off = base + i * _GK
            pltpu.sync_copy(dst_hbm.at[pl.ds(off, _GK)], idx_v)
            pltpu.async_copy(table_hbm.at[idx_v], rows_v, sem).wait()
            pltpu.sync_copy(rows_v, xi_hbm.at[pl.ds(off, _GK)])
            pltpu.sync_copy(src_hbm.at[pl.ds(off, _GK)], idx_v)
            pltpu.async_copy(table_hbm.at[idx_v], rows_v, sem).wait()
            pltpu.sync_copy(rows_v, xj_hbm.at[pl.ds(off, _GK)])
            return 0

        lax.fori_loop(0, nch, body, 0)

    return k


_gather128 = _make_gather(128)


_CK = 640
_NCOLG = 8
def _make_segmax():
    EC = E // NSUB
    nch = EC // _CK

    @functools.partial(
        pl.kernel,
        mesh=plsc.VectorSubcoreMesh(**_MESH),
        compiler_params=pltpu.CompilerParams(needs_layout_passes=False),
        out_type=jax.ShapeDtypeStruct((NSUB, 64, N), jnp.float32),
        scratch_types=[
            pltpu.VMEM((_CK,), jnp.int32),
            pltpu.VMEM((8, _CK), jnp.float32),
            pltpu.VMEM((8, N), jnp.float32),
            pltpu.VMEM((16,), jnp.int32),
            pltpu.VMEM((16,), jnp.float32),
        ],
    )
    def k(h2t_hbm, dst_hbm, neg_hbm, out_hbm, dstbuf, h2buf, acc, dtmp, vtmp):
        wid = lax.axis_index("s") * _NC + lax.axis_index("c")
        p = wid % _NCOLG
        q = wid // _NCOLG
        pltpu.sync_copy(neg_hbm, acc)
        iota = lax.iota(jnp.int32, 16)
        colv = iota & 7
        rowsel = iota >> 3
        perm8 = iota ^ 8

        def chunk(i, _):
            eoff = q * EC + i * _CK
            pltpu.sync_copy(dst_hbm.at[pl.ds(eoff, _CK)], dstbuf)
            pltpu.sync_copy(h2t_hbm.at[pl.ds(p * 8, 8), pl.ds(eoff, _CK)],
                            h2buf)

            def grp(m, _):
                for j in range(8):
                    rows = m * 16 + 2 * j + rowsel
                    dperm = plsc.load_gather(dstbuf, [rows])
                    vals = plsc.load_gather(h2buf, [colv, rows])
                    dtmp[...] = dperm
                    vtmp[...] = vals
                    drot = plsc.load_gather(dtmp, [perm8])
                    vrot = plsc.load_gather(vtmp, [perm8])
                    vals = jnp.where(dperm == drot,
                                     jnp.maximum(vals, vrot), vals)
                    old = plsc.load_gather(acc, [colv, dperm])
                    plsc.store_scatter(acc, [colv, dperm],
                                       jnp.maximum(old, vals))
                return 0

            lax.fori_loop(0, _CK // 16, grp, 0)
            return 0

        lax.fori_loop(0, nch, chunk, 0)
        pltpu.sync_copy(acc, out_hbm.at[q, pl.ds(p * 8, 8), :])

    return k


_segmax = _make_segmax()


def _gather_rows2(table, idx_dst, idx_src):
    return _gather128(table, idx_dst, idx_src)


def _segmax_partials(h2t, dst, neg):
    return jnp.transpose(_segmax(h2t, dst, neg), (0, 2, 1))


def kernel(x, edge_index, batch, W1a, b1a, W2a, b2a, g1, be1, W1b, b1b, W2b, b2b,
           g2, be2, Wp1, bp1, gp, bep, Wp2, bp2):
    src = edge_index[0]
    dst = edge_index[1]
    neg = jnp.full((8, N), NEG, jnp.float32)

    def layer(h, W1, b1, W2, b2, gamma, beta, pad):
        xi, xj = _gather_rows2(h, dst, src)
        h2t = _edge_mlp(xi, xj, W1, b1, W2, b2)
        partials = _segmax_partials(h2t, dst, neg)
        maxed, stats = _combine_stats(partials)
        return _bn_apply(maxed, stats, gamma, beta, pad)

    h = layer(x, W1a, b1a, W2a, b2a, g1, be1, True)
    x_node = layer(h, W1b, b1b, W2b, b2b, g2, be2, False)

    cnt, sums, maxs = _pool_acc(x_node, batch.reshape(N, 1))
    z_proj, x_graph = _final(cnt, sums, maxs, Wp1, bp1, gp, bep, Wp2, bp2)
    return (z_proj, x_node, x_graph)


# speedup vs baseline: 1.6995x; 1.2850x over previous
import functools

import jax
import jax.numpy as jnp
from jax import lax
from jax.experimental import pallas as pl
from jax.experimental.pallas import tpu as pltpu
from jax.experimental.pallas import tpu_sc as plsc

N = 10000
E = 320000
G = 64
NSUB = 4
EBS = 1280
NBS = 2000
NEG = float("-inf")


def _edge_mlp_body(Din, xi_ref, xj_ref, w1t_ref, w1b_ref, b1_ref, w2_ref, b2_ref, o_ref):
    xi = xi_ref[...][:, :Din]
    xj = xj_ref[...][:, :Din]
    h = jnp.dot(xi, w1t_ref[...]) + jnp.dot(xj - xi, w1b_ref[...]) + b1_ref[...]
    h = jnp.maximum(h, 0.0)
    h = jnp.dot(h, w2_ref[...]) + b2_ref[...]
    o_ref[...] = lax.transpose(jnp.maximum(h, 0.0), (1, 0))


def _edge_mlp(xi, xj, W1, b1, W2, b2):
    Dpad = xi.shape[1]
    D = W1.shape[0] // 2
    H = W2.shape[1]
    nb = E // EBS
    return pl.pallas_call(
        functools.partial(_edge_mlp_body, D),
        grid=(nb,),
        in_specs=[
            pl.BlockSpec((EBS, Dpad), lambda i: (i, 0)),
            pl.BlockSpec((EBS, Dpad), lambda i: (i, 0)),
            pl.BlockSpec((D, H), lambda i: (0, 0)),
            pl.BlockSpec((D, H), lambda i: (0, 0)),
            pl.BlockSpec((1, H), lambda i: (0, 0)),
            pl.BlockSpec((H, H), lambda i: (0, 0)),
            pl.BlockSpec((1, H), lambda i: (0, 0)),
        ],
        out_specs=pl.BlockSpec((H, EBS), lambda i: (0, i)),
        out_shape=jax.ShapeDtypeStruct((H, E), jnp.float32),
    )(xi, xj, W1[:D], W1[D:], b1.reshape(1, H), W2, b2.reshape(1, H))


def _combine_stats_body(p_ref, maxed_ref, stats_ref):
    i = pl.program_id(0)
    m = p_ref[0]
    for q in range(1, NSUB):
        m = jnp.maximum(m, p_ref[q])
    m = jnp.where(jnp.isfinite(m), m, 0.0)
    maxed_ref[...] = m

    @pl.when(i == 0)
    def _():
        stats_ref[...] = jnp.zeros_like(stats_ref)

    s = jnp.sum(m, axis=0, keepdims=True)
    ss = jnp.sum(m * m, axis=0, keepdims=True)
    stats_ref[...] += jnp.concatenate([s, ss], axis=0)


def _combine_stats(partials):
    H = partials.shape[2]
    nb = N // NBS
    return pl.pallas_call(
        _combine_stats_body,
        grid=(nb,),
        in_specs=[pl.BlockSpec((NSUB, NBS, H), lambda i: (0, i, 0))],
        out_specs=[
            pl.BlockSpec((NBS, H), lambda i: (i, 0)),
            pl.BlockSpec((2, H), lambda i: (0, 0)),
        ],
        out_shape=[
            jax.ShapeDtypeStruct((N, H), jnp.float32),
            jax.ShapeDtypeStruct((2, H), jnp.float32),
        ],
    )(partials)


def _bn_apply_body(pad, x_ref, stats_ref, g_ref, be_ref, o_ref):
    s = stats_ref[0:1]
    ss = stats_ref[1:2]
    mu = s / float(N)
    var = ss / float(N) - mu * mu
    rstd = lax.rsqrt(var + 1e-5)
    y = (x_ref[...] - mu) * rstd * g_ref[...] + be_ref[...]
    if pad:
        y = jnp.concatenate([y, jnp.zeros_like(y)], axis=1)
    o_ref[...] = y


def _bn_apply(x, stats, gamma, beta, pad):
    H = x.shape[1]
    Ho = 2 * H if pad else H
    nb = N // NBS
    return pl.pallas_call(
        functools.partial(_bn_apply_body, pad),
        grid=(nb,),
        in_specs=[
            pl.BlockSpec((NBS, H), lambda i: (i, 0)),
            pl.BlockSpec((2, H), lambda i: (0, 0)),
            pl.BlockSpec((1, H), lambda i: (0, 0)),
            pl.BlockSpec((1, H), lambda i: (0, 0)),
        ],
        out_specs=pl.BlockSpec((NBS, Ho), lambda i: (i, 0)),
        out_shape=jax.ShapeDtypeStruct((N, Ho), jnp.float32),
    )(x, stats, gamma.reshape(1, H), beta.reshape(1, H))


def _pool_acc_body(xn_ref, b_ref, cnt_ref, sum_ref, max_ref):
    i = pl.program_id(0)

    @pl.when(i == 0)
    def _():
        cnt_ref[...] = jnp.zeros_like(cnt_ref)
        sum_ref[...] = jnp.zeros_like(sum_ref)
        max_ref[...] = jnp.full_like(max_ref, NEG)

    xn = xn_ref[...]
    b = b_ref[...]
    iota = lax.broadcasted_iota(jnp.int32, (NBS, G), 1)
    mf = (b == iota).astype(jnp.float32)
    cnt_ref[...] += jnp.sum(mf, axis=0, keepdims=True)
    sum_ref[...] += lax.dot_general(
        mf, xn, (((0,), (0,)), ((), ())), precision=lax.Precision.HIGHEST
    )

    def body(g, _):
        mask = b == g
        mg = jnp.max(jnp.where(mask, xn, NEG), axis=0, keepdims=True)
        max_ref[pl.ds(g, 1), :] = jnp.maximum(max_ref[pl.ds(g, 1), :], mg)
        return 0

    lax.fori_loop(0, G, body, 0)


def _pool_acc(x_node, batch_col):
    H = x_node.shape[1]
    nb = N // NBS
    return pl.pallas_call(
        _pool_acc_body,
        grid=(nb,),
        in_specs=[
            pl.BlockSpec((NBS, H), lambda i: (i, 0)),
            pl.BlockSpec((NBS, 1), lambda i: (i, 0)),
        ],
        out_specs=[
            pl.BlockSpec((1, G), lambda i: (0, 0)),
            pl.BlockSpec((G, H), lambda i: (0, 0)),
            pl.BlockSpec((G, H), lambda i: (0, 0)),
        ],
        out_shape=[
            jax.ShapeDtypeStruct((1, G), jnp.float32),
            jax.ShapeDtypeStruct((G, H), jnp.float32),
            jax.ShapeDtypeStruct((G, H), jnp.float32),
        ],
    )(x_node, batch_col)


def _final_body(cnt_ref, sum_ref, max_ref, wp1_ref, bp1_ref, gp_ref, bep_ref,
                wp2_ref, bp2_ref, zp_ref, xg_ref):
    cnt = jnp.maximum(cnt_ref[...], 1.0)
    mean_pool = sum_ref[...] / cnt.reshape(G, 1)
    mx = max_ref[...]
    max_pool = jnp.where(jnp.isfinite(mx), mx, 0.0)
    x_graph = jnp.concatenate([mean_pool, max_pool], axis=1)
    xg_ref[...] = x_graph
    z = jnp.dot(x_graph, wp1_ref[...]) + bp1_ref[...]
    mu = jnp.mean(z, axis=0, keepdims=True)
    var = jnp.mean((z - mu) * (z - mu), axis=0, keepdims=True)
    z = (z - mu) * lax.rsqrt(var + 1e-5) * gp_ref[...] + bep_ref[...]
    z = jnp.maximum(z, 0.0)
    zp_ref[...] = jnp.dot(z, wp2_ref[...]) + bp2_ref[...]


def _final(cnt, sums, maxs, Wp1, bp1, gp, bep, Wp2, bp2):
    H = sums.shape[1]
    P = Wp1.shape[1]
    return pl.pallas_call(
        _final_body,
        out_shape=[
            jax.ShapeDtypeStruct((G, P), jnp.float32),
            jax.ShapeDtypeStruct((G, 2 * H), jnp.float32),
        ],
    )(cnt, sums, maxs, Wp1, bp1.reshape(1, P), gp.reshape(1, P),
      bep.reshape(1, P), Wp2, bp2.reshape(1, P))


_NC = 2
_NS = 16
_NW = _NC * _NS
_GK = 40
_MESH = dict(core_axis_name="c", subcore_axis_name="s")


def _make_gather(Dh):
    per_w = E // _NW
    nch = per_w // _GK

    @functools.partial(
        pl.kernel,
        mesh=plsc.VectorSubcoreMesh(**_MESH),
        out_type=[
            jax.ShapeDtypeStruct((E, Dh), jnp.float32),
            jax.ShapeDtypeStruct((E, Dh), jnp.float32),
        ],
        scratch_types=[
            pltpu.VMEM((per_w,), jnp.int32),
            pltpu.VMEM((_GK, Dh), jnp.float32),
            pltpu.VMEM((_GK, Dh), jnp.float32),
            pltpu.SemaphoreType.DMA,
            pltpu.SemaphoreType.DMA,
            pltpu.SemaphoreType.DMA,
            pltpu.SemaphoreType.DMA,
        ],
    )
    def k(table_hbm, dst_hbm, src_hbm, xi_hbm, xj_hbm,
          idxb, r0, r1, g0, g1, w0, w1):
        wid = lax.axis_index("s") * _NC + lax.axis_index("c")
        base = wid * per_w

        def drain(sem, buf):
            pltpu.make_async_copy(table_hbm.at[pl.ds(0, _GK)], buf, sem).wait()

        def stream(idx_hbm, out_hbm):
            pltpu.sync_copy(idx_hbm.at[pl.ds(base, per_w)], idxb)
            pltpu.async_copy(table_hbm.at[idxb.at[pl.ds(0, _GK)]], r0, g0)

            def pair(ip, _):
                i0 = 2 * ip

                @pl.when(ip > 0)
                def _():
                    drain(w1, r1)
                pltpu.async_copy(
                    table_hbm.at[idxb.at[pl.ds((i0 + 1) * _GK, _GK)]], r1, g1)
                drain(g0, r0)
                pltpu.async_copy(r0, out_hbm.at[pl.ds(base + i0 * _GK, _GK)], w0)

                @pl.when(i0 + 2 < nch)
                def _():
                    drain(w0, r0)
                    pltpu.async_copy(
                        table_hbm.at[idxb.at[pl.ds((i0 + 2) * _GK, _GK)]], r0, g0)
                drain(g1, r1)
                pltpu.async_copy(
                    r1, out_hbm.at[pl.ds(base + (i0 + 1) * _GK, _GK)], w1)
                return 0

            lax.fori_loop(0, nch // 2, pair, 0)
            drain(w0, r0)
            drain(w1, r1)

        stream(dst_hbm, xi_hbm)
        stream(src_hbm, xj_hbm)

    return k


_gather128 = _make_gather(128)


_CK = 3200
_PERM_DNUMS = lax.GatherDimensionNumbers(
    offset_dims=(), collapsed_slice_dims=(0,), start_index_map=(0,))


def _vperm(v, perm):
    return lax.gather(v, perm[:, None], _PERM_DNUMS, (1,),
                      mode=lax.GatherScatterMode.PROMISE_IN_BOUNDS)
_NCOLG = 8
def _make_segmax():
    EC = E // NSUB
    nch = EC // _CK

    @functools.partial(
        pl.kernel,
        mesh=plsc.VectorSubcoreMesh(**_MESH),
        compiler_params=pltpu.CompilerParams(needs_layout_passes=False),
        out_type=jax.ShapeDtypeStruct((NSUB, 64, N), jnp.float32),
        scratch_types=[
            pltpu.VMEM((_CK,), jnp.int32),
            pltpu.VMEM((8, _CK), jnp.float32),
            pltpu.VMEM((8, N), jnp.float32),
        ],
    )
    def k(h2t_hbm, dst_hbm, neg_hbm, out_hbm, dstbuf, h2buf, acc):
        wid = lax.axis_index("s") * _NC + lax.axis_index("c")
        p = wid % _NCOLG
        q = wid // _NCOLG
        pltpu.sync_copy(neg_hbm, acc)
        iota = lax.iota(jnp.int32, 16)
        colv = iota & 7
        rowsel = iota >> 3
        perm8 = iota ^ 8

        def chunk(i, _):
            eoff = q * EC + i * _CK
            pltpu.sync_copy(dst_hbm.at[pl.ds(eoff, _CK)], dstbuf)
            pltpu.sync_copy(h2t_hbm.at[pl.ds(p * 8, 8), pl.ds(eoff, _CK)],
                            h2buf)

            def grp(m, _):
                for j in range(8):
                    rows = m * 16 + 2 * j + rowsel
                    dperm = plsc.load_gather(dstbuf, [rows])
                    vals = plsc.load_gather(h2buf, [colv, rows])
                    drot = _vperm(dperm, perm8)
                    vrot = _vperm(vals, perm8)
                    vals = jnp.where(dperm == drot,
                                     jnp.maximum(vals, vrot), vals)
                    old = plsc.load_gather(acc, [colv, dperm])
                    plsc.store_scatter(acc, [colv, dperm],
                                       jnp.maximum(old, vals))
                return 0

            lax.fori_loop(0, _CK // 16, grp, 0)
            return 0

        lax.fori_loop(0, nch, chunk, 0)
        pltpu.sync_copy(acc, out_hbm.at[q, pl.ds(p * 8, 8), :])

    return k


_segmax = _make_segmax()


def _gather_rows2(table, idx_dst, idx_src):
    return _gather128(table, idx_dst, idx_src)


def _segmax_partials(h2t, dst, neg):
    return jnp.transpose(_segmax(h2t, dst, neg), (0, 2, 1))


def kernel(x, edge_index, batch, W1a, b1a, W2a, b2a, g1, be1, W1b, b1b, W2b, b2b,
           g2, be2, Wp1, bp1, gp, bep, Wp2, bp2):
    src = edge_index[0]
    dst = edge_index[1]
    neg = jnp.full((8, N), NEG, jnp.float32)

    def layer(h, W1, b1, W2, b2, gamma, beta, pad):
        xi, xj = _gather_rows2(h, dst, src)
        h2t = _edge_mlp(xi, xj, W1, b1, W2, b2)
        partials = _segmax_partials(h2t, dst, neg)
        maxed, stats = _combine_stats(partials)
        return _bn_apply(maxed, stats, gamma, beta, pad)

    h = layer(x, W1a, b1a, W2a, b2a, g1, be1, True)
    x_node = layer(h, W1b, b1b, W2b, b2b, g2, be2, False)

    cnt, sums, maxs = _pool_acc(x_node, batch.reshape(N, 1))
    z_proj, x_graph = _final(cnt, sums, maxs, Wp1, bp1, gp, bep, Wp2, bp2)
    return (z_proj, x_node, x_graph)


# speedup vs baseline: 1.8516x; 1.0895x over previous
import functools

import jax
import jax.numpy as jnp
from jax import lax
from jax.experimental import pallas as pl
from jax.experimental.pallas import tpu as pltpu
from jax.experimental.pallas import tpu_sc as plsc

N = 10000
E = 320000
G = 64
NSUB = 4
EBS = 1280
NBS = 2000
NEG = float("-inf")


def _edge_mlp_body(Din, xi_ref, xj_ref, w1t_ref, w1b_ref, b1_ref, w2_ref, b2_ref, o_ref):
    xi = xi_ref[...][:, :Din]
    xj = xj_ref[...][:, :Din]
    h = jnp.dot(xi, w1t_ref[...]) + jnp.dot(xj - xi, w1b_ref[...]) + b1_ref[...]
    h = jnp.maximum(h, 0.0)
    h = jnp.dot(h, w2_ref[...]) + b2_ref[...]
    o_ref[...] = lax.transpose(jnp.maximum(h, 0.0), (1, 0))


def _edge_mlp(xi, xj, W1, b1, W2, b2):
    Dpad = xi.shape[1]
    Eh = xi.shape[0]
    D = W1.shape[0] // 2
    H = W2.shape[1]
    nb = Eh // EBS
    return pl.pallas_call(
        functools.partial(_edge_mlp_body, D),
        grid=(nb,),
        in_specs=[
            pl.BlockSpec((EBS, Dpad), lambda i: (i, 0)),
            pl.BlockSpec((EBS, Dpad), lambda i: (i, 0)),
            pl.BlockSpec((D, H), lambda i: (0, 0)),
            pl.BlockSpec((D, H), lambda i: (0, 0)),
            pl.BlockSpec((1, H), lambda i: (0, 0)),
            pl.BlockSpec((H, H), lambda i: (0, 0)),
            pl.BlockSpec((1, H), lambda i: (0, 0)),
        ],
        out_specs=pl.BlockSpec((H, EBS), lambda i: (0, i)),
        out_shape=jax.ShapeDtypeStruct((H, Eh), jnp.float32),
    )(xi, xj, W1[:D], W1[D:], b1.reshape(1, H), W2, b2.reshape(1, H))


def _combine_stats_body(nparts, p_ref, maxed_ref, stats_ref):
    i = pl.program_id(0)
    m = p_ref[0]
    for q in range(1, nparts):
        m = jnp.maximum(m, p_ref[q])
    m = jnp.where(jnp.isfinite(m), m, 0.0)
    maxed_ref[...] = m

    @pl.when(i == 0)
    def _():
        stats_ref[...] = jnp.zeros_like(stats_ref)

    s = jnp.sum(m, axis=0, keepdims=True)
    ss = jnp.sum(m * m, axis=0, keepdims=True)
    stats_ref[...] += jnp.concatenate([s, ss], axis=0)


def _combine_stats(partials):
    nparts = partials.shape[0]
    H = partials.shape[2]
    nb = N // NBS
    return pl.pallas_call(
        functools.partial(_combine_stats_body, nparts),
        grid=(nb,),
        in_specs=[pl.BlockSpec((nparts, NBS, H), lambda i: (0, i, 0))],
        out_specs=[
            pl.BlockSpec((NBS, H), lambda i: (i, 0)),
            pl.BlockSpec((2, H), lambda i: (0, 0)),
        ],
        out_shape=[
            jax.ShapeDtypeStruct((N, H), jnp.float32),
            jax.ShapeDtypeStruct((2, H), jnp.float32),
        ],
    )(partials)


def _bn_apply_body(pad, x_ref, stats_ref, g_ref, be_ref, o_ref):
    s = stats_ref[0:1]
    ss = stats_ref[1:2]
    mu = s / float(N)
    var = ss / float(N) - mu * mu
    rstd = lax.rsqrt(var + 1e-5)
    y = (x_ref[...] - mu) * rstd * g_ref[...] + be_ref[...]
    if pad:
        y = jnp.concatenate([y, jnp.zeros_like(y)], axis=1)
    o_ref[...] = y


def _bn_apply(x, stats, gamma, beta, pad):
    H = x.shape[1]
    Ho = 2 * H if pad else H
    nb = N // NBS
    return pl.pallas_call(
        functools.partial(_bn_apply_body, pad),
        grid=(nb,),
        in_specs=[
            pl.BlockSpec((NBS, H), lambda i: (i, 0)),
            pl.BlockSpec((2, H), lambda i: (0, 0)),
            pl.BlockSpec((1, H), lambda i: (0, 0)),
            pl.BlockSpec((1, H), lambda i: (0, 0)),
        ],
        out_specs=pl.BlockSpec((NBS, Ho), lambda i: (i, 0)),
        out_shape=jax.ShapeDtypeStruct((N, Ho), jnp.float32),
    )(x, stats, gamma.reshape(1, H), beta.reshape(1, H))


def _pool_acc_body(xn_ref, b_ref, cnt_ref, sum_ref, max_ref):
    i = pl.program_id(0)

    @pl.when(i == 0)
    def _():
        cnt_ref[...] = jnp.zeros_like(cnt_ref)
        sum_ref[...] = jnp.zeros_like(sum_ref)
        max_ref[...] = jnp.full_like(max_ref, NEG)

    xn = xn_ref[...]
    b = b_ref[...]
    iota = lax.broadcasted_iota(jnp.int32, (NBS, G), 1)
    mf = (b == iota).astype(jnp.float32)
    cnt_ref[...] += jnp.sum(mf, axis=0, keepdims=True)
    sum_ref[...] += lax.dot_general(
        mf, xn, (((0,), (0,)), ((), ())), precision=lax.Precision.HIGHEST
    )

    def body(g, _):
        mask = b == g
        mg = jnp.max(jnp.where(mask, xn, NEG), axis=0, keepdims=True)
        max_ref[pl.ds(g, 1), :] = jnp.maximum(max_ref[pl.ds(g, 1), :], mg)
        return 0

    lax.fori_loop(0, G, body, 0)


def _pool_acc(x_node, batch_col):
    H = x_node.shape[1]
    nb = N // NBS
    return pl.pallas_call(
        _pool_acc_body,
        grid=(nb,),
        in_specs=[
            pl.BlockSpec((NBS, H), lambda i: (i, 0)),
            pl.BlockSpec((NBS, 1), lambda i: (i, 0)),
        ],
        out_specs=[
            pl.BlockSpec((1, G), lambda i: (0, 0)),
            pl.BlockSpec((G, H), lambda i: (0, 0)),
            pl.BlockSpec((G, H), lambda i: (0, 0)),
        ],
        out_shape=[
            jax.ShapeDtypeStruct((1, G), jnp.float32),
            jax.ShapeDtypeStruct((G, H), jnp.float32),
            jax.ShapeDtypeStruct((G, H), jnp.float32),
        ],
    )(x_node, batch_col)


def _final_body(cnt_ref, sum_ref, max_ref, wp1_ref, bp1_ref, gp_ref, bep_ref,
                wp2_ref, bp2_ref, zp_ref, xg_ref):
    cnt = jnp.maximum(cnt_ref[...], 1.0)
    mean_pool = sum_ref[...] / cnt.reshape(G, 1)
    mx = max_ref[...]
    max_pool = jnp.where(jnp.isfinite(mx), mx, 0.0)
    x_graph = jnp.concatenate([mean_pool, max_pool], axis=1)
    xg_ref[...] = x_graph
    z = jnp.dot(x_graph, wp1_ref[...]) + bp1_ref[...]
    mu = jnp.mean(z, axis=0, keepdims=True)
    var = jnp.mean((z - mu) * (z - mu), axis=0, keepdims=True)
    z = (z - mu) * lax.rsqrt(var + 1e-5) * gp_ref[...] + bep_ref[...]
    z = jnp.maximum(z, 0.0)
    zp_ref[...] = jnp.dot(z, wp2_ref[...]) + bp2_ref[...]


def _final(cnt, sums, maxs, Wp1, bp1, gp, bep, Wp2, bp2):
    H = sums.shape[1]
    P = Wp1.shape[1]
    return pl.pallas_call(
        _final_body,
        out_shape=[
            jax.ShapeDtypeStruct((G, P), jnp.float32),
            jax.ShapeDtypeStruct((G, 2 * H), jnp.float32),
        ],
    )(cnt, sums, maxs, Wp1, bp1.reshape(1, P), gp.reshape(1, P),
      bep.reshape(1, P), Wp2, bp2.reshape(1, P))


_NC = 2
_NS = 16
_NW = _NC * _NS
_GK = 40
_MESH = dict(core_axis_name="c", subcore_axis_name="s")


def _make_gather(Dh, Eh):
    per_w = Eh // _NW
    nch = per_w // _GK

    @functools.partial(
        pl.kernel,
        mesh=plsc.VectorSubcoreMesh(**_MESH),
        out_type=[
            jax.ShapeDtypeStruct((Eh, Dh), jnp.float32),
            jax.ShapeDtypeStruct((Eh, Dh), jnp.float32),
        ],
        scratch_types=[
            pltpu.VMEM((per_w,), jnp.int32),
            pltpu.VMEM((_GK, Dh), jnp.float32),
            pltpu.VMEM((_GK, Dh), jnp.float32),
            pltpu.SemaphoreType.DMA,
            pltpu.SemaphoreType.DMA,
            pltpu.SemaphoreType.DMA,
            pltpu.SemaphoreType.DMA,
        ],
    )
    def k(table_hbm, dst_hbm, src_hbm, xi_hbm, xj_hbm,
          idxb, r0, r1, g0, g1, w0, w1):
        wid = lax.axis_index("s") * _NC + lax.axis_index("c")
        base = wid * per_w

        def drain(sem, buf):
            pltpu.make_async_copy(table_hbm.at[pl.ds(0, _GK)], buf, sem).wait()

        def stream(idx_hbm, out_hbm):
            pltpu.sync_copy(idx_hbm.at[pl.ds(base, per_w)], idxb)
            pltpu.async_copy(table_hbm.at[idxb.at[pl.ds(0, _GK)]], r0, g0)

            def pair(ip, _):
                i0 = 2 * ip

                @pl.when(ip > 0)
                def _():
                    drain(w1, r1)
                pltpu.async_copy(
                    table_hbm.at[idxb.at[pl.ds((i0 + 1) * _GK, _GK)]], r1, g1)
                drain(g0, r0)
                pltpu.async_copy(r0, out_hbm.at[pl.ds(base + i0 * _GK, _GK)], w0)

                @pl.when(i0 + 2 < nch)
                def _():
                    drain(w0, r0)
                    pltpu.async_copy(
                        table_hbm.at[idxb.at[pl.ds((i0 + 2) * _GK, _GK)]], r0, g0)
                drain(g1, r1)
                pltpu.async_copy(
                    r1, out_hbm.at[pl.ds(base + (i0 + 1) * _GK, _GK)], w1)
                return 0

            lax.fori_loop(0, nch // 2, pair, 0)
            drain(w0, r0)
            drain(w1, r1)

        stream(dst_hbm, xi_hbm)
        stream(src_hbm, xj_hbm)

    return k


_EA = 163840
_EB = E - _EA
_gather_A = _make_gather(128, _EA)
_gather_B = _make_gather(128, _EB)


_PERM_DNUMS = lax.GatherDimensionNumbers(
    offset_dims=(), collapsed_slice_dims=(0,), start_index_map=(0,))


def _vperm(v, perm):
    return lax.gather(v, perm[:, None], _PERM_DNUMS, (1,),
                      mode=lax.GatherScatterMode.PROMISE_IN_BOUNDS)
_NCOLG = 8
def _make_segmax(Eh, _CK):
    EC = Eh // NSUB
    nch = EC // _CK

    @functools.partial(
        pl.kernel,
        mesh=plsc.VectorSubcoreMesh(**_MESH),
        compiler_params=pltpu.CompilerParams(needs_layout_passes=False),
        out_type=jax.ShapeDtypeStruct((NSUB, 64, N), jnp.float32),
        scratch_types=[
            pltpu.VMEM((_CK,), jnp.int32),
            pltpu.VMEM((8, _CK), jnp.float32),
            pltpu.VMEM((8, N), jnp.float32),
        ],
    )
    def k(h2t_hbm, dst_hbm, neg_hbm, out_hbm, dstbuf, h2buf, acc):
        wid = lax.axis_index("s") * _NC + lax.axis_index("c")
        p = wid % _NCOLG
        q = wid // _NCOLG
        pltpu.sync_copy(neg_hbm, acc)
        iota = lax.iota(jnp.int32, 16)
        colv = iota & 7
        rowsel = iota >> 3
        perm8 = iota ^ 8

        def chunk(i, _):
            eoff = q * EC + i * _CK
            pltpu.sync_copy(dst_hbm.at[pl.ds(eoff, _CK)], dstbuf)
            pltpu.sync_copy(h2t_hbm.at[pl.ds(p * 8, 8), pl.ds(eoff, _CK)],
                            h2buf)

            def grp(m, _):
                for j in range(8):
                    rows = m * 16 + 2 * j + rowsel
                    dperm = plsc.load_gather(dstbuf, [rows])
                    vals = plsc.load_gather(h2buf, [colv, rows])
                    drot = _vperm(dperm, perm8)
                    vrot = _vperm(vals, perm8)
                    vals = jnp.where(dperm == drot,
                                     jnp.maximum(vals, vrot), vals)
                    old = plsc.load_gather(acc, [colv, dperm])
                    plsc.store_scatter(acc, [colv, dperm],
                                       jnp.maximum(old, vals))
                return 0

            lax.fori_loop(0, _CK // 16, grp, 0)
            return 0

        lax.fori_loop(0, nch, chunk, 0)
        pltpu.sync_copy(acc, out_hbm.at[q, pl.ds(p * 8, 8), :])

    return k


_segmax_A = _make_segmax(_EA, 2560)
_segmax_B = _make_segmax(_EB, 640)


def kernel(x, edge_index, batch, W1a, b1a, W2a, b2a, g1, be1, W1b, b1b, W2b, b2b,
           g2, be2, Wp1, bp1, gp, bep, Wp2, bp2):
    src = edge_index[0]
    dst = edge_index[1]
    neg = jnp.full((8, N), NEG, jnp.float32)
    halves = (
        (dst[:_EA], src[:_EA], _gather_A, _segmax_A),
        (dst[_EA:], src[_EA:], _gather_B, _segmax_B),
    )

    def layer(h, W1, b1, W2, b2, gamma, beta, pad):
        parts = []
        for dsth, srch, gat, smx in halves:
            xi, xj = gat(h, dsth, srch)
            h2t = _edge_mlp(xi, xj, W1, b1, W2, b2)
            parts.append(jnp.transpose(smx(h2t, dsth, neg), (0, 2, 1)))
        maxed, stats = _combine_stats(jnp.concatenate(parts, axis=0))
        return _bn_apply(maxed, stats, gamma, beta, pad)

    h = layer(x, W1a, b1a, W2a, b2a, g1, be1, True)
    x_node = layer(h, W1b, b1b, W2b, b2b, g2, be2, False)

    cnt, sums, maxs = _pool_acc(x_node, batch.reshape(N, 1))
    z_proj, x_graph = _final(cnt, sums, maxs, Wp1, bp1, gp, bep, Wp2, bp2)
    return (z_proj, x_node, x_graph)


# speedup vs baseline: 1.9437x; 1.0497x over previous
import functools

import jax
import jax.numpy as jnp
from jax import lax
from jax.experimental import pallas as pl
from jax.experimental.pallas import tpu as pltpu
from jax.experimental.pallas import tpu_sc as plsc

N = 10000
E = 320000
G = 64
NSUB = 4
EBS = 1280
NBS = 2000
NEG = float("-inf")


def _edge_mlp_body(Din, xi_ref, xj_ref, w1t_ref, w1b_ref, b1_ref, w2_ref, b2_ref, o_ref):
    xi = xi_ref[...][:, :Din]
    xj = xj_ref[...][:, :Din]
    h = jnp.dot(xi, w1t_ref[...]) + jnp.dot(xj - xi, w1b_ref[...]) + b1_ref[...]
    h = jnp.maximum(h, 0.0)
    h = jnp.dot(h, w2_ref[...]) + b2_ref[...]
    o_ref[...] = lax.transpose(jnp.maximum(h, 0.0), (1, 0))


def _edge_mlp(xi, xj, W1, b1, W2, b2):
    Dpad = xi.shape[1]
    Eh = xi.shape[0]
    D = W1.shape[0] // 2
    H = W2.shape[1]
    nb = Eh // EBS
    return pl.pallas_call(
        functools.partial(_edge_mlp_body, D),
        grid=(nb,),
        in_specs=[
            pl.BlockSpec((EBS, Dpad), lambda i: (i, 0)),
            pl.BlockSpec((EBS, Dpad), lambda i: (i, 0)),
            pl.BlockSpec((D, H), lambda i: (0, 0)),
            pl.BlockSpec((D, H), lambda i: (0, 0)),
            pl.BlockSpec((1, H), lambda i: (0, 0)),
            pl.BlockSpec((H, H), lambda i: (0, 0)),
            pl.BlockSpec((1, H), lambda i: (0, 0)),
        ],
        out_specs=pl.BlockSpec((H, EBS), lambda i: (0, i)),
        out_shape=jax.ShapeDtypeStruct((H, Eh), jnp.float32),
    )(xi, xj, W1[:D], W1[D:], b1.reshape(1, H), W2, b2.reshape(1, H))


def _combine_stats_body(nparts, p_ref, maxed_ref, stats_ref):
    i = pl.program_id(0)
    m = p_ref[0]
    for q in range(1, nparts):
        m = jnp.maximum(m, p_ref[q])
    m = jnp.where(jnp.isfinite(m), m, 0.0)
    maxed_ref[...] = m

    @pl.when(i == 0)
    def _():
        stats_ref[...] = jnp.zeros_like(stats_ref)

    s = jnp.sum(m, axis=0, keepdims=True)
    ss = jnp.sum(m * m, axis=0, keepdims=True)
    stats_ref[...] += jnp.concatenate([s, ss], axis=0)


def _combine_stats(partials):
    nparts = partials.shape[0]
    H = partials.shape[2]
    nb = N // NBS
    return pl.pallas_call(
        functools.partial(_combine_stats_body, nparts),
        grid=(nb,),
        in_specs=[pl.BlockSpec((nparts, NBS, H), lambda i: (0, i, 0))],
        out_specs=[
            pl.BlockSpec((NBS, H), lambda i: (i, 0)),
            pl.BlockSpec((2, H), lambda i: (0, 0)),
        ],
        out_shape=[
            jax.ShapeDtypeStruct((N, H), jnp.float32),
            jax.ShapeDtypeStruct((2, H), jnp.float32),
        ],
    )(partials)


def _bn_apply_body(pad, x_ref, stats_ref, g_ref, be_ref, o_ref):
    s = stats_ref[0:1]
    ss = stats_ref[1:2]
    mu = s / float(N)
    var = ss / float(N) - mu * mu
    rstd = lax.rsqrt(var + 1e-5)
    y = (x_ref[...] - mu) * rstd * g_ref[...] + be_ref[...]
    if pad:
        y = jnp.concatenate([y, jnp.zeros_like(y)], axis=1)
    o_ref[...] = y


def _bn_apply(x, stats, gamma, beta, pad):
    H = x.shape[1]
    Ho = 2 * H if pad else H
    nb = N // NBS
    return pl.pallas_call(
        functools.partial(_bn_apply_body, pad),
        grid=(nb,),
        in_specs=[
            pl.BlockSpec((NBS, H), lambda i: (i, 0)),
            pl.BlockSpec((2, H), lambda i: (0, 0)),
            pl.BlockSpec((1, H), lambda i: (0, 0)),
            pl.BlockSpec((1, H), lambda i: (0, 0)),
        ],
        out_specs=pl.BlockSpec((NBS, Ho), lambda i: (i, 0)),
        out_shape=jax.ShapeDtypeStruct((N, Ho), jnp.float32),
    )(x, stats, gamma.reshape(1, H), beta.reshape(1, H))


def _pool_acc_body(xn_ref, b_ref, cnt_ref, sum_ref, max_ref):
    i = pl.program_id(0)

    @pl.when(i == 0)
    def _():
        cnt_ref[...] = jnp.zeros_like(cnt_ref)
        sum_ref[...] = jnp.zeros_like(sum_ref)
        max_ref[...] = jnp.full_like(max_ref, NEG)

    xn = xn_ref[...]
    b = b_ref[...]
    iota = lax.broadcasted_iota(jnp.int32, (NBS, G), 1)
    mf = (b == iota).astype(jnp.float32)
    cnt_ref[...] += jnp.sum(mf, axis=0, keepdims=True)
    sum_ref[...] += lax.dot_general(
        mf, xn, (((0,), (0,)), ((), ())), precision=lax.Precision.HIGHEST
    )

    def body(g, _):
        mask = b == g
        mg = jnp.max(jnp.where(mask, xn, NEG), axis=0, keepdims=True)
        max_ref[pl.ds(g, 1), :] = jnp.maximum(max_ref[pl.ds(g, 1), :], mg)
        return 0

    lax.fori_loop(0, G, body, 0)


def _pool_acc(x_node, batch_col):
    H = x_node.shape[1]
    nb = N // NBS
    return pl.pallas_call(
        _pool_acc_body,
        grid=(nb,),
        in_specs=[
            pl.BlockSpec((NBS, H), lambda i: (i, 0)),
            pl.BlockSpec((NBS, 1), lambda i: (i, 0)),
        ],
        out_specs=[
            pl.BlockSpec((1, G), lambda i: (0, 0)),
            pl.BlockSpec((G, H), lambda i: (0, 0)),
            pl.BlockSpec((G, H), lambda i: (0, 0)),
        ],
        out_shape=[
            jax.ShapeDtypeStruct((1, G), jnp.float32),
            jax.ShapeDtypeStruct((G, H), jnp.float32),
            jax.ShapeDtypeStruct((G, H), jnp.float32),
        ],
    )(x_node, batch_col)


def _final_body(cnt_ref, sum_ref, max_ref, wp1_ref, bp1_ref, gp_ref, bep_ref,
                wp2_ref, bp2_ref, zp_ref, xg_ref):
    cnt = jnp.maximum(cnt_ref[...], 1.0)
    mean_pool = sum_ref[...] / cnt.reshape(G, 1)
    mx = max_ref[...]
    max_pool = jnp.where(jnp.isfinite(mx), mx, 0.0)
    x_graph = jnp.concatenate([mean_pool, max_pool], axis=1)
    xg_ref[...] = x_graph
    z = jnp.dot(x_graph, wp1_ref[...]) + bp1_ref[...]
    mu = jnp.mean(z, axis=0, keepdims=True)
    var = jnp.mean((z - mu) * (z - mu), axis=0, keepdims=True)
    z = (z - mu) * lax.rsqrt(var + 1e-5) * gp_ref[...] + bep_ref[...]
    z = jnp.maximum(z, 0.0)
    zp_ref[...] = jnp.dot(z, wp2_ref[...]) + bp2_ref[...]


def _final(cnt, sums, maxs, Wp1, bp1, gp, bep, Wp2, bp2):
    H = sums.shape[1]
    P = Wp1.shape[1]
    return pl.pallas_call(
        _final_body,
        out_shape=[
            jax.ShapeDtypeStruct((G, P), jnp.float32),
            jax.ShapeDtypeStruct((G, 2 * H), jnp.float32),
        ],
    )(cnt, sums, maxs, Wp1, bp1.reshape(1, P), gp.reshape(1, P),
      bep.reshape(1, P), Wp2, bp2.reshape(1, P))


_NC = 2
_NS = 16
_NW = _NC * _NS
_GK = 40
_MESH = dict(core_axis_name="c", subcore_axis_name="s")


def _make_gather(Dh, Eh):
    per_w = Eh // _NW
    nch = per_w // _GK

    @functools.partial(
        pl.kernel,
        mesh=plsc.VectorSubcoreMesh(**_MESH),
        out_type=[
            jax.ShapeDtypeStruct((Eh, Dh), jnp.float32),
            jax.ShapeDtypeStruct((Eh, Dh), jnp.float32),
        ],
        scratch_types=[
            pltpu.VMEM((per_w,), jnp.int32),
            pltpu.VMEM((_GK, Dh), jnp.float32),
            pltpu.VMEM((_GK, Dh), jnp.float32),
            pltpu.SemaphoreType.DMA,
            pltpu.SemaphoreType.DMA,
            pltpu.SemaphoreType.DMA,
            pltpu.SemaphoreType.DMA,
        ],
    )
    def k(table_hbm, dst_hbm, src_hbm, xi_hbm, xj_hbm,
          idxb, r0, r1, g0, g1, w0, w1):
        wid = lax.axis_index("s") * _NC + lax.axis_index("c")
        base = wid * per_w

        def drain(sem, buf):
            pltpu.make_async_copy(table_hbm.at[pl.ds(0, _GK)], buf, sem).wait()

        def stream(idx_hbm, out_hbm):
            pltpu.sync_copy(idx_hbm.at[pl.ds(base, per_w)], idxb)
            pltpu.async_copy(table_hbm.at[idxb.at[pl.ds(0, _GK)]], r0, g0)

            def pair(ip, _):
                i0 = 2 * ip

                @pl.when(ip > 0)
                def _():
                    drain(w1, r1)
                pltpu.async_copy(
                    table_hbm.at[idxb.at[pl.ds((i0 + 1) * _GK, _GK)]], r1, g1)
                drain(g0, r0)
                pltpu.async_copy(r0, out_hbm.at[pl.ds(base + i0 * _GK, _GK)], w0)

                @pl.when(i0 + 2 < nch)
                def _():
                    drain(w0, r0)
                    pltpu.async_copy(
                        table_hbm.at[idxb.at[pl.ds((i0 + 2) * _GK, _GK)]], r0, g0)
                drain(g1, r1)
                pltpu.async_copy(
                    r1, out_hbm.at[pl.ds(base + (i0 + 1) * _GK, _GK)], w1)
                return 0

            lax.fori_loop(0, nch // 2, pair, 0)
            drain(w0, r0)
            drain(w1, r1)

        stream(dst_hbm, xi_hbm)
        stream(src_hbm, xj_hbm)

    return k


_EA = 163840
_EB = E - _EA
_gather_A = _make_gather(128, _EA)
_gather_B = _make_gather(128, _EB)


_PERM_DNUMS = lax.GatherDimensionNumbers(
    offset_dims=(), collapsed_slice_dims=(0,), start_index_map=(0,))


def _vperm(v, perm):
    return lax.gather(v, perm[:, None], _PERM_DNUMS, (1,),
                      mode=lax.GatherScatterMode.PROMISE_IN_BOUNDS)
_NCOLG = 8
def _make_segmax(Eh, _CK):
    EC = Eh // NSUB
    nch = EC // _CK

    @functools.partial(
        pl.kernel,
        mesh=plsc.VectorSubcoreMesh(**_MESH),
        compiler_params=pltpu.CompilerParams(needs_layout_passes=False),
        out_type=jax.ShapeDtypeStruct((NSUB, 64 * N), jnp.float32),
        scratch_types=[
            pltpu.VMEM((_CK,), jnp.int32),
            pltpu.VMEM((8, _CK), jnp.float32),
            pltpu.VMEM((8 * N,), jnp.float32),
        ],
    )
    def k(h2t_hbm, dst_hbm, neg_hbm, out_hbm, dstbuf, h2buf, acc):
        wid = lax.axis_index("s") * _NC + lax.axis_index("c")
        p = wid % _NCOLG
        q = wid // _NCOLG
        pltpu.sync_copy(neg_hbm, acc)
        iota = lax.iota(jnp.int32, 16)
        colv = iota & 7
        colN = colv * N
        rowsel = iota >> 3
        perm8 = iota ^ 8

        def chunk(i, _):
            eoff = q * EC + i * _CK
            pltpu.sync_copy(dst_hbm.at[pl.ds(eoff, _CK)], dstbuf)
            pltpu.sync_copy(h2t_hbm.at[pl.ds(p * 8, 8), pl.ds(eoff, _CK)],
                            h2buf)

            def grp(m, _):
                for j in range(8):
                    rows = m * 16 + 2 * j + rowsel
                    dperm = plsc.load_gather(dstbuf, [rows])
                    vals = plsc.load_gather(h2buf, [colv, rows])
                    drot = _vperm(dperm, perm8)
                    vrot = _vperm(vals, perm8)
                    vals = jnp.where(dperm == drot,
                                     jnp.maximum(vals, vrot), vals)
                    addr = colN + dperm
                    old = plsc.load_gather(acc, [addr])
                    plsc.store_scatter(acc, [addr],
                                       jnp.maximum(old, vals))
                return 0

            lax.fori_loop(0, _CK // 16, grp, 0)
            return 0

        lax.fori_loop(0, nch, chunk, 0)
        pltpu.sync_copy(acc, out_hbm.at[q, pl.ds(p * 8 * N, 8 * N)])

    return k


_segmax_A = _make_segmax(_EA, 2560)
_segmax_B = _make_segmax(_EB, 640)


def kernel(x, edge_index, batch, W1a, b1a, W2a, b2a, g1, be1, W1b, b1b, W2b, b2b,
           g2, be2, Wp1, bp1, gp, bep, Wp2, bp2):
    src = edge_index[0]
    dst = edge_index[1]
    neg = jnp.full((8 * N,), NEG, jnp.float32)
    halves = (
        (dst[:_EA], src[:_EA], _gather_A, _segmax_A),
        (dst[_EA:], src[_EA:], _gather_B, _segmax_B),
    )

    def layer(h, W1, b1, W2, b2, gamma, beta, pad):
        parts = []
        for dsth, srch, gat, smx in halves:
            xi, xj = gat(h, dsth, srch)
            h2t = _edge_mlp(xi, xj, W1, b1, W2, b2)
            pr = smx(h2t, dsth, neg).reshape(NSUB, 8, 8, N)
            parts.append(jnp.transpose(pr, (0, 3, 1, 2)).reshape(NSUB, N, 64))
        maxed, stats = _combine_stats(jnp.concatenate(parts, axis=0))
        return _bn_apply(maxed, stats, gamma, beta, pad)

    h = layer(x, W1a, b1a, W2a, b2a, g1, be1, True)
    x_node = layer(h, W1b, b1b, W2b, b2b, g2, be2, False)

    cnt, sums, maxs = _pool_acc(x_node, batch.reshape(N, 1))
    z_proj, x_graph = _final(cnt, sums, maxs, Wp1, bp1, gp, bep, Wp2, bp2)
    return (z_proj, x_node, x_graph)


# speedup vs baseline: 2.1527x; 1.1076x over previous
import functools

import jax
import jax.numpy as jnp
from jax import lax
from jax.experimental import pallas as pl
from jax.experimental.pallas import tpu as pltpu
from jax.experimental.pallas import tpu_sc as plsc

N = 10000
E = 320000
G = 64
NSUB = 4
EBS = 1280
NBS = 2000
NEG = float("-inf")


def _edge_mlp_body(Din, xi_ref, xj_ref, w1t_ref, w1b_ref, b1_ref, w2_ref, b2_ref, o_ref):
    xi = xi_ref[...][:, :Din]
    xj = xj_ref[...][:, :Din]
    h = jnp.dot(xi, w1t_ref[...]) + jnp.dot(xj - xi, w1b_ref[...]) + b1_ref[...]
    h = jnp.maximum(h, 0.0)
    h = jnp.dot(h, w2_ref[...]) + b2_ref[...]
    o_ref[...] = lax.transpose(jnp.maximum(h, 0.0), (1, 0))


def _edge_mlp(xi, xj, W1, b1, W2, b2):
    Dpad = xi.shape[1]
    Eh = xi.shape[0]
    D = W1.shape[0] // 2
    H = W2.shape[1]
    nb = Eh // EBS
    return pl.pallas_call(
        functools.partial(_edge_mlp_body, D),
        grid=(nb,),
        in_specs=[
            pl.BlockSpec((EBS, Dpad), lambda i: (i, 0)),
            pl.BlockSpec((EBS, Dpad), lambda i: (i, 0)),
            pl.BlockSpec((D, H), lambda i: (0, 0)),
            pl.BlockSpec((D, H), lambda i: (0, 0)),
            pl.BlockSpec((1, H), lambda i: (0, 0)),
            pl.BlockSpec((H, H), lambda i: (0, 0)),
            pl.BlockSpec((1, H), lambda i: (0, 0)),
        ],
        out_specs=pl.BlockSpec((H, EBS), lambda i: (0, i)),
        out_shape=jax.ShapeDtypeStruct((H, Eh), jnp.float32),
    )(xi, xj, W1[:D], W1[D:], b1.reshape(1, H), W2, b2.reshape(1, H))


def _combine_stats_body(nparts, p_ref, maxed_ref, stats_ref):
    i = pl.program_id(0)
    m = p_ref[0]
    for q in range(1, nparts):
        m = jnp.maximum(m, p_ref[q])
    m = jnp.where(jnp.isfinite(m), m, 0.0)
    maxed_ref[...] = m

    @pl.when(i == 0)
    def _():
        stats_ref[...] = jnp.zeros_like(stats_ref)

    s = jnp.sum(m, axis=0, keepdims=True)
    ss = jnp.sum(m * m, axis=0, keepdims=True)
    stats_ref[...] += jnp.concatenate([s, ss], axis=0)


def _combine_stats(partials):
    nparts = partials.shape[0]
    H = partials.shape[2]
    nb = N // NBS
    return pl.pallas_call(
        functools.partial(_combine_stats_body, nparts),
        grid=(nb,),
        in_specs=[pl.BlockSpec((nparts, NBS, H), lambda i: (0, i, 0))],
        out_specs=[
            pl.BlockSpec((NBS, H), lambda i: (i, 0)),
            pl.BlockSpec((2, H), lambda i: (0, 0)),
        ],
        out_shape=[
            jax.ShapeDtypeStruct((N, H), jnp.float32),
            jax.ShapeDtypeStruct((2, H), jnp.float32),
        ],
    )(partials)


def _bn_apply_body(pad, x_ref, stats_ref, g_ref, be_ref, o_ref):
    s = stats_ref[0:1]
    ss = stats_ref[1:2]
    mu = s / float(N)
    var = ss / float(N) - mu * mu
    rstd = lax.rsqrt(var + 1e-5)
    y = (x_ref[...] - mu) * rstd * g_ref[...] + be_ref[...]
    if pad:
        y = jnp.concatenate([y, jnp.zeros_like(y)], axis=1)
    o_ref[...] = y


def _bn_apply(x, stats, gamma, beta, pad):
    H = x.shape[1]
    Ho = 2 * H if pad else H
    nb = N // NBS
    return pl.pallas_call(
        functools.partial(_bn_apply_body, pad),
        grid=(nb,),
        in_specs=[
            pl.BlockSpec((NBS, H), lambda i: (i, 0)),
            pl.BlockSpec((2, H), lambda i: (0, 0)),
            pl.BlockSpec((1, H), lambda i: (0, 0)),
            pl.BlockSpec((1, H), lambda i: (0, 0)),
        ],
        out_specs=pl.BlockSpec((NBS, Ho), lambda i: (i, 0)),
        out_shape=jax.ShapeDtypeStruct((N, Ho), jnp.float32),
    )(x, stats, gamma.reshape(1, H), beta.reshape(1, H))


def _pool_acc_body(xn_ref, b_ref, cnt_ref, sum_ref, max_ref):
    i = pl.program_id(0)

    @pl.when(i == 0)
    def _():
        cnt_ref[...] = jnp.zeros_like(cnt_ref)
        sum_ref[...] = jnp.zeros_like(sum_ref)
        max_ref[...] = jnp.full_like(max_ref, NEG)

    xn = xn_ref[...]
    b = b_ref[...]
    iota = lax.broadcasted_iota(jnp.int32, (NBS, G), 1)
    mf = (b == iota).astype(jnp.float32)
    cnt_ref[...] += jnp.sum(mf, axis=0, keepdims=True)
    sum_ref[...] += lax.dot_general(
        mf, xn, (((0,), (0,)), ((), ())), precision=lax.Precision.HIGHEST
    )

    def body(g, _):
        mask = b == g
        mg = jnp.max(jnp.where(mask, xn, NEG), axis=0, keepdims=True)
        max_ref[pl.ds(g, 1), :] = jnp.maximum(max_ref[pl.ds(g, 1), :], mg)
        return 0

    lax.fori_loop(0, G, body, 0)


def _pool_acc(x_node, batch_col):
    H = x_node.shape[1]
    nb = N // NBS
    return pl.pallas_call(
        _pool_acc_body,
        grid=(nb,),
        in_specs=[
            pl.BlockSpec((NBS, H), lambda i: (i, 0)),
            pl.BlockSpec((NBS, 1), lambda i: (i, 0)),
        ],
        out_specs=[
            pl.BlockSpec((1, G), lambda i: (0, 0)),
            pl.BlockSpec((G, H), lambda i: (0, 0)),
            pl.BlockSpec((G, H), lambda i: (0, 0)),
        ],
        out_shape=[
            jax.ShapeDtypeStruct((1, G), jnp.float32),
            jax.ShapeDtypeStruct((G, H), jnp.float32),
            jax.ShapeDtypeStruct((G, H), jnp.float32),
        ],
    )(x_node, batch_col)


def _final_body(cnt_ref, sum_ref, max_ref, wp1_ref, bp1_ref, gp_ref, bep_ref,
                wp2_ref, bp2_ref, zp_ref, xg_ref):
    cnt = jnp.maximum(cnt_ref[...], 1.0)
    mean_pool = sum_ref[...] / cnt.reshape(G, 1)
    mx = max_ref[...]
    max_pool = jnp.where(jnp.isfinite(mx), mx, 0.0)
    x_graph = jnp.concatenate([mean_pool, max_pool], axis=1)
    xg_ref[...] = x_graph
    z = jnp.dot(x_graph, wp1_ref[...]) + bp1_ref[...]
    mu = jnp.mean(z, axis=0, keepdims=True)
    var = jnp.mean((z - mu) * (z - mu), axis=0, keepdims=True)
    z = (z - mu) * lax.rsqrt(var + 1e-5) * gp_ref[...] + bep_ref[...]
    z = jnp.maximum(z, 0.0)
    zp_ref[...] = jnp.dot(z, wp2_ref[...]) + bp2_ref[...]


def _final(cnt, sums, maxs, Wp1, bp1, gp, bep, Wp2, bp2):
    H = sums.shape[1]
    P = Wp1.shape[1]
    return pl.pallas_call(
        _final_body,
        out_shape=[
            jax.ShapeDtypeStruct((G, P), jnp.float32),
            jax.ShapeDtypeStruct((G, 2 * H), jnp.float32),
        ],
    )(cnt, sums, maxs, Wp1, bp1.reshape(1, P), gp.reshape(1, P),
      bep.reshape(1, P), Wp2, bp2.reshape(1, P))


_NC = 2
_NS = 16
_NW = _NC * _NS
_GK = 40
_MESH = dict(core_axis_name="c", subcore_axis_name="s")


def _make_gather(Dh, Eh):
    per_w = Eh // _NW
    nch = per_w // _GK

    @functools.partial(
        pl.kernel,
        mesh=plsc.VectorSubcoreMesh(**_MESH),
        out_type=[
            jax.ShapeDtypeStruct((Eh, Dh), jnp.float32),
            jax.ShapeDtypeStruct((Eh, Dh), jnp.float32),
        ],
        scratch_types=[
            pltpu.VMEM((per_w,), jnp.int32),
            pltpu.VMEM((_GK, Dh), jnp.float32),
            pltpu.VMEM((_GK, Dh), jnp.float32),
            pltpu.SemaphoreType.DMA,
            pltpu.SemaphoreType.DMA,
            pltpu.SemaphoreType.DMA,
            pltpu.SemaphoreType.DMA,
        ],
    )
    def k(table_hbm, dst_hbm, src_hbm, xi_hbm, xj_hbm,
          idxb, r0, r1, g0, g1, w0, w1):
        wid = lax.axis_index("s") * _NC + lax.axis_index("c")
        base = wid * per_w

        def drain(sem, buf):
            pltpu.make_async_copy(table_hbm.at[pl.ds(0, _GK)], buf, sem).wait()

        def stream(idx_hbm, out_hbm):
            pltpu.sync_copy(idx_hbm.at[pl.ds(base, per_w)], idxb)
            pltpu.async_copy(table_hbm.at[idxb.at[pl.ds(0, _GK)]], r0, g0)

            def pair(ip, _):
                i0 = 2 * ip

                @pl.when(ip > 0)
                def _():
                    drain(w1, r1)
                pltpu.async_copy(
                    table_hbm.at[idxb.at[pl.ds((i0 + 1) * _GK, _GK)]], r1, g1)
                drain(g0, r0)
                pltpu.async_copy(r0, out_hbm.at[pl.ds(base + i0 * _GK, _GK)], w0)

                @pl.when(i0 + 2 < nch)
                def _():
                    drain(w0, r0)
                    pltpu.async_copy(
                        table_hbm.at[idxb.at[pl.ds((i0 + 2) * _GK, _GK)]], r0, g0)
                drain(g1, r1)
                pltpu.async_copy(
                    r1, out_hbm.at[pl.ds(base + (i0 + 1) * _GK, _GK)], w1)
                return 0

            lax.fori_loop(0, nch // 2, pair, 0)
            drain(w0, r0)
            drain(w1, r1)

        stream(dst_hbm, xi_hbm)
        stream(src_hbm, xj_hbm)

    return k


_EA = 163840
_EB = E - _EA
_gather_A = _make_gather(128, _EA)
_gather_B = _make_gather(128, _EB)


_PERM_DNUMS = lax.GatherDimensionNumbers(
    offset_dims=(), collapsed_slice_dims=(0,), start_index_map=(0,))


def _vperm(v, perm):
    return lax.gather(v, perm[:, None], _PERM_DNUMS, (1,),
                      mode=lax.GatherScatterMode.PROMISE_IN_BOUNDS)
_NCOLG = 8
def _make_segmax(Eh, _CK):
    EC = Eh // NSUB
    nch = EC // _CK

    @functools.partial(
        pl.kernel,
        mesh=plsc.VectorSubcoreMesh(**_MESH),
        compiler_params=pltpu.CompilerParams(needs_layout_passes=False),
        out_type=jax.ShapeDtypeStruct((NSUB, 64 * N), jnp.float32),
        scratch_types=[
            pltpu.VMEM((_CK,), jnp.int32),
            pltpu.VMEM((8, _CK), jnp.float32),
            pltpu.VMEM((8 * N,), jnp.float32),
        ],
    )
    def k(h2t_hbm, dst_hbm, neg_hbm, out_hbm, dstbuf, h2buf, acc):
        wid = lax.axis_index("s") * _NC + lax.axis_index("c")
        p = wid % _NCOLG
        q = wid // _NCOLG
        pltpu.sync_copy(neg_hbm, acc)
        iota = lax.iota(jnp.int32, 16)
        colv = iota & 7
        colN = colv * N
        rowsel = iota >> 3
        perm8 = iota ^ 8

        def chunk(i, _):
            eoff = q * EC + i * _CK
            pltpu.sync_copy(dst_hbm.at[pl.ds(eoff, _CK)], dstbuf)
            pltpu.sync_copy(h2t_hbm.at[pl.ds(p * 8, 8), pl.ds(eoff, _CK)],
                            h2buf)

            def grp(m, _):
                dvec = dstbuf[pl.ds(m * 16, 16)]
                for j in range(8):
                    rows = m * 16 + 2 * j + rowsel
                    dperm = _vperm(dvec, 2 * j + rowsel)
                    vals = plsc.load_gather(h2buf, [colv, rows])
                    drot = _vperm(dperm, perm8)
                    vrot = _vperm(vals, perm8)
                    vals = jnp.where(dperm == drot,
                                     jnp.maximum(vals, vrot), vals)
                    addr = colN + dperm
                    old = plsc.load_gather(acc, [addr])
                    plsc.store_scatter(acc, [addr],
                                       jnp.maximum(old, vals))
                return 0

            lax.fori_loop(0, _CK // 16, grp, 0)
            return 0

        lax.fori_loop(0, nch, chunk, 0)
        pltpu.sync_copy(acc, out_hbm.at[q, pl.ds(p * 8 * N, 8 * N)])

    return k


_segmax_A = _make_segmax(_EA, 2560)
_segmax_B = _make_segmax(_EB, 640)


def kernel(x, edge_index, batch, W1a, b1a, W2a, b2a, g1, be1, W1b, b1b, W2b, b2b,
           g2, be2, Wp1, bp1, gp, bep, Wp2, bp2):
    src = edge_index[0]
    dst = edge_index[1]
    neg = jnp.full((8 * N,), NEG, jnp.float32)
    halves = (
        (dst[:_EA], src[:_EA], _gather_A, _segmax_A),
        (dst[_EA:], src[_EA:], _gather_B, _segmax_B),
    )

    def layer(h, W1, b1, W2, b2, gamma, beta, pad):
        parts = []
        for dsth, srch, gat, smx in halves:
            xi, xj = gat(h, dsth, srch)
            h2t = _edge_mlp(xi, xj, W1, b1, W2, b2)
            pr = smx(h2t, dsth, neg).reshape(NSUB, 8, 8, N)
            parts.append(jnp.transpose(pr, (0, 3, 1, 2)).reshape(NSUB, N, 64))
        maxed, stats = _combine_stats(jnp.concatenate(parts, axis=0))
        return _bn_apply(maxed, stats, gamma, beta, pad)

    h = layer(x, W1a, b1a, W2a, b2a, g1, be1, True)
    x_node = layer(h, W1b, b1b, W2b, b2b, g2, be2, False)

    cnt, sums, maxs = _pool_acc(x_node, batch.reshape(N, 1))
    z_proj, x_graph = _final(cnt, sums, maxs, Wp1, bp1, gp, bep, Wp2, bp2)
    return (z_proj, x_node, x_graph)


# speedup vs baseline: 2.2604x; 1.0500x over previous
import functools

import jax
import jax.numpy as jnp
from jax import lax
from jax.experimental import pallas as pl
from jax.experimental.pallas import tpu as pltpu
from jax.experimental.pallas import tpu_sc as plsc

N = 10000
E = 320000
G = 64
NSUB = 4
EBS = 1280
NBS = 2000
NEG = float("-inf")


def _edge_mlp_body(Din, xi_ref, xj_ref, w1t_ref, w1b_ref, b1_ref, w2_ref, b2_ref, o_ref):
    xi = xi_ref[...][:, :Din]
    xj = xj_ref[...][:, :Din]
    h = jnp.dot(xi, w1t_ref[...]) + jnp.dot(xj - xi, w1b_ref[...]) + b1_ref[...]
    h = jnp.maximum(h, 0.0)
    h = jnp.dot(h, w2_ref[...]) + b2_ref[...]
    o_ref[...] = lax.transpose(jnp.maximum(h, 0.0), (1, 0))


def _edge_mlp(xi, xj, W1, b1, W2, b2):
    Dpad = xi.shape[1]
    Eh = xi.shape[0]
    D = W1.shape[0] // 2
    H = W2.shape[1]
    nb = Eh // EBS
    return pl.pallas_call(
        functools.partial(_edge_mlp_body, D),
        grid=(nb,),
        in_specs=[
            pl.BlockSpec((EBS, Dpad), lambda i: (i, 0)),
            pl.BlockSpec((EBS, Dpad), lambda i: (i, 0)),
            pl.BlockSpec((D, H), lambda i: (0, 0)),
            pl.BlockSpec((D, H), lambda i: (0, 0)),
            pl.BlockSpec((1, H), lambda i: (0, 0)),
            pl.BlockSpec((H, H), lambda i: (0, 0)),
            pl.BlockSpec((1, H), lambda i: (0, 0)),
        ],
        out_specs=pl.BlockSpec((H, EBS), lambda i: (0, i)),
        out_shape=jax.ShapeDtypeStruct((H, Eh), jnp.float32),
    )(xi, xj, W1[:D], W1[D:], b1.reshape(1, H), W2, b2.reshape(1, H))


def _combine_stats_body(nparts, p_ref, maxed_ref, stats_ref):
    i = pl.program_id(0)
    m = p_ref[0]
    for q in range(1, nparts):
        m = jnp.maximum(m, p_ref[q])
    m = jnp.where(jnp.isfinite(m), m, 0.0)
    maxed_ref[...] = m

    @pl.when(i == 0)
    def _():
        stats_ref[...] = jnp.zeros_like(stats_ref)

    s = jnp.sum(m, axis=0, keepdims=True)
    ss = jnp.sum(m * m, axis=0, keepdims=True)
    stats_ref[...] += jnp.concatenate([s, ss], axis=0)


def _combine_stats(partials):
    nparts = partials.shape[0]
    H = partials.shape[2]
    nb = N // NBS
    return pl.pallas_call(
        functools.partial(_combine_stats_body, nparts),
        grid=(nb,),
        in_specs=[pl.BlockSpec((nparts, NBS, H), lambda i: (0, i, 0))],
        out_specs=[
            pl.BlockSpec((NBS, H), lambda i: (i, 0)),
            pl.BlockSpec((2, H), lambda i: (0, 0)),
        ],
        out_shape=[
            jax.ShapeDtypeStruct((N, H), jnp.float32),
            jax.ShapeDtypeStruct((2, H), jnp.float32),
        ],
    )(partials)


def _bn_apply_body(pad, x_ref, stats_ref, g_ref, be_ref, o_ref):
    s = stats_ref[0:1]
    ss = stats_ref[1:2]
    mu = s / float(N)
    var = ss / float(N) - mu * mu
    rstd = lax.rsqrt(var + 1e-5)
    y = (x_ref[...] - mu) * rstd * g_ref[...] + be_ref[...]
    if pad:
        y = jnp.concatenate([y, jnp.zeros_like(y)], axis=1)
    o_ref[...] = y


def _bn_apply(x, stats, gamma, beta, pad):
    H = x.shape[1]
    Ho = 2 * H if pad else H
    nb = N // NBS
    return pl.pallas_call(
        functools.partial(_bn_apply_body, pad),
        grid=(nb,),
        in_specs=[
            pl.BlockSpec((NBS, H), lambda i: (i, 0)),
            pl.BlockSpec((2, H), lambda i: (0, 0)),
            pl.BlockSpec((1, H), lambda i: (0, 0)),
            pl.BlockSpec((1, H), lambda i: (0, 0)),
        ],
        out_specs=pl.BlockSpec((NBS, Ho), lambda i: (i, 0)),
        out_shape=jax.ShapeDtypeStruct((N, Ho), jnp.float32),
    )(x, stats, gamma.reshape(1, H), beta.reshape(1, H))


def _pool_acc_body(xn_ref, b_ref, cnt_ref, sum_ref, max_ref):
    i = pl.program_id(0)

    @pl.when(i == 0)
    def _():
        cnt_ref[...] = jnp.zeros_like(cnt_ref)
        sum_ref[...] = jnp.zeros_like(sum_ref)
        max_ref[...] = jnp.full_like(max_ref, NEG)

    xn = xn_ref[...]
    b = b_ref[...]
    iota = lax.broadcasted_iota(jnp.int32, (NBS, G), 1)
    mf = (b == iota).astype(jnp.float32)
    cnt_ref[...] += jnp.sum(mf, axis=0, keepdims=True)
    sum_ref[...] += lax.dot_general(
        mf, xn, (((0,), (0,)), ((), ())), precision=lax.Precision.HIGHEST
    )

    def body(g, _):
        mask = b == g
        mg = jnp.max(jnp.where(mask, xn, NEG), axis=0, keepdims=True)
        max_ref[pl.ds(g, 1), :] = jnp.maximum(max_ref[pl.ds(g, 1), :], mg)
        return 0

    lax.fori_loop(0, G, body, 0)


def _pool_acc(x_node, batch_col):
    H = x_node.shape[1]
    nb = N // NBS
    return pl.pallas_call(
        _pool_acc_body,
        grid=(nb,),
        in_specs=[
            pl.BlockSpec((NBS, H), lambda i: (i, 0)),
            pl.BlockSpec((NBS, 1), lambda i: (i, 0)),
        ],
        out_specs=[
            pl.BlockSpec((1, G), lambda i: (0, 0)),
            pl.BlockSpec((G, H), lambda i: (0, 0)),
            pl.BlockSpec((G, H), lambda i: (0, 0)),
        ],
        out_shape=[
            jax.ShapeDtypeStruct((1, G), jnp.float32),
            jax.ShapeDtypeStruct((G, H), jnp.float32),
            jax.ShapeDtypeStruct((G, H), jnp.float32),
        ],
    )(x_node, batch_col)


def _final_body(cnt_ref, sum_ref, max_ref, wp1_ref, bp1_ref, gp_ref, bep_ref,
                wp2_ref, bp2_ref, zp_ref, xg_ref):
    cnt = jnp.maximum(cnt_ref[...], 1.0)
    mean_pool = sum_ref[...] / cnt.reshape(G, 1)
    mx = max_ref[...]
    max_pool = jnp.where(jnp.isfinite(mx), mx, 0.0)
    x_graph = jnp.concatenate([mean_pool, max_pool], axis=1)
    xg_ref[...] = x_graph
    z = jnp.dot(x_graph, wp1_ref[...]) + bp1_ref[...]
    mu = jnp.mean(z, axis=0, keepdims=True)
    var = jnp.mean((z - mu) * (z - mu), axis=0, keepdims=True)
    z = (z - mu) * lax.rsqrt(var + 1e-5) * gp_ref[...] + bep_ref[...]
    z = jnp.maximum(z, 0.0)
    zp_ref[...] = jnp.dot(z, wp2_ref[...]) + bp2_ref[...]


def _final(cnt, sums, maxs, Wp1, bp1, gp, bep, Wp2, bp2):
    H = sums.shape[1]
    P = Wp1.shape[1]
    return pl.pallas_call(
        _final_body,
        out_shape=[
            jax.ShapeDtypeStruct((G, P), jnp.float32),
            jax.ShapeDtypeStruct((G, 2 * H), jnp.float32),
        ],
    )(cnt, sums, maxs, Wp1, bp1.reshape(1, P), gp.reshape(1, P),
      bep.reshape(1, P), Wp2, bp2.reshape(1, P))


_NC = 2
_NS = 16
_NW = _NC * _NS
_GK = 80
_MESH = dict(core_axis_name="c", subcore_axis_name="s")


def _make_gather(Dh, Eh):
    per_w = Eh // _NW
    nch = per_w // _GK

    @functools.partial(
        pl.kernel,
        mesh=plsc.VectorSubcoreMesh(**_MESH),
        out_type=[
            jax.ShapeDtypeStruct((Eh, Dh), jnp.float32),
            jax.ShapeDtypeStruct((Eh, Dh), jnp.float32),
        ],
        scratch_types=[
            pltpu.VMEM((per_w,), jnp.int32),
            pltpu.VMEM((_GK, Dh), jnp.float32),
            pltpu.VMEM((_GK, Dh), jnp.float32),
            pltpu.SemaphoreType.DMA,
            pltpu.SemaphoreType.DMA,
            pltpu.SemaphoreType.DMA,
            pltpu.SemaphoreType.DMA,
        ],
    )
    def k(table_hbm, dst_hbm, src_hbm, xi_hbm, xj_hbm,
          idxb, r0, r1, g0, g1, w0, w1):
        wid = lax.axis_index("s") * _NC + lax.axis_index("c")
        base = wid * per_w

        def drain(sem, buf):
            pltpu.make_async_copy(table_hbm.at[pl.ds(0, _GK)], buf, sem).wait()

        def stream(idx_hbm, out_hbm):
            pltpu.sync_copy(idx_hbm.at[pl.ds(base, per_w)], idxb)
            pltpu.async_copy(table_hbm.at[idxb.at[pl.ds(0, _GK)]], r0, g0)

            def pair(ip, _):
                i0 = 2 * ip

                @pl.when(ip > 0)
                def _():
                    drain(w1, r1)
                pltpu.async_copy(
                    table_hbm.at[idxb.at[pl.ds((i0 + 1) * _GK, _GK)]], r1, g1)
                drain(g0, r0)
                pltpu.async_copy(r0, out_hbm.at[pl.ds(base + i0 * _GK, _GK)], w0)

                @pl.when(i0 + 2 < nch)
                def _():
                    drain(w0, r0)
                    pltpu.async_copy(
                        table_hbm.at[idxb.at[pl.ds((i0 + 2) * _GK, _GK)]], r0, g0)
                drain(g1, r1)
                pltpu.async_copy(
                    r1, out_hbm.at[pl.ds(base + (i0 + 1) * _GK, _GK)], w1)
                return 0

            lax.fori_loop(0, nch // 2, pair, 0)
            if nch % 2:
                drain(g0, r0)
                pltpu.async_copy(
                    r0, out_hbm.at[pl.ds(base + (nch - 1) * _GK, _GK)], w0)
            drain(w0, r0)
            drain(w1, r1)

        stream(dst_hbm, xi_hbm)
        stream(src_hbm, xj_hbm)

    return k


_EA = 163840
_EB = E - _EA
_gather_A = _make_gather(128, _EA)
_gather_B = _make_gather(128, _EB)


_PERM_DNUMS = lax.GatherDimensionNumbers(
    offset_dims=(), collapsed_slice_dims=(0,), start_index_map=(0,))


def _vperm(v, perm):
    return lax.gather(v, perm[:, None], _PERM_DNUMS, (1,),
                      mode=lax.GatherScatterMode.PROMISE_IN_BOUNDS)
_NCOLG = 8
def _make_segmax(Eh, _CK):
    EC = Eh // NSUB
    nch = EC // _CK

    @functools.partial(
        pl.kernel,
        mesh=plsc.VectorSubcoreMesh(**_MESH),
        compiler_params=pltpu.CompilerParams(needs_layout_passes=False),
        out_type=jax.ShapeDtypeStruct((NSUB, 64 * N), jnp.float32),
        scratch_types=[
            pltpu.VMEM((_CK,), jnp.int32),
            pltpu.VMEM((8, _CK), jnp.float32),
            pltpu.VMEM((8 * N,), jnp.float32),
        ],
    )
    def k(h2t_hbm, dst_hbm, neg_hbm, out_hbm, dstbuf, h2buf, acc):
        wid = lax.axis_index("s") * _NC + lax.axis_index("c")
        p = wid % _NCOLG
        q = wid // _NCOLG
        pltpu.sync_copy(neg_hbm, acc)
        iota = lax.iota(jnp.int32, 16)
        colv = iota & 7
        colN = colv * N
        rowsel = iota >> 3
        perm8 = iota ^ 8

        def chunk(i, _):
            eoff = q * EC + i * _CK
            pltpu.sync_copy(dst_hbm.at[pl.ds(eoff, _CK)], dstbuf)
            pltpu.sync_copy(h2t_hbm.at[pl.ds(p * 8, 8), pl.ds(eoff, _CK)],
                            h2buf)

            def grp(m, _):
                dvec = dstbuf[pl.ds(m * 16, 16)]
                for j in range(8):
                    rows = m * 16 + 2 * j + rowsel
                    dperm = _vperm(dvec, 2 * j + rowsel)
                    vals = plsc.load_gather(h2buf, [colv, rows])
                    drot = _vperm(dperm, perm8)
                    vrot = _vperm(vals, perm8)
                    vals = jnp.where(dperm == drot,
                                     jnp.maximum(vals, vrot), vals)
                    addr = colN + dperm
                    old = plsc.load_gather(acc, [addr])
                    plsc.store_scatter(acc, [addr],
                                       jnp.maximum(old, vals))
                return 0

            lax.fori_loop(0, _CK // 16, grp, 0)
            return 0

        lax.fori_loop(0, nch, chunk, 0)
        pltpu.sync_copy(acc, out_hbm.at[q, pl.ds(p * 8 * N, 8 * N)])

    return k


_segmax_A = _make_segmax(_EA, 2560)
_segmax_B = _make_segmax(_EB, 640)


def kernel(x, edge_index, batch, W1a, b1a, W2a, b2a, g1, be1, W1b, b1b, W2b, b2b,
           g2, be2, Wp1, bp1, gp, bep, Wp2, bp2):
    src = edge_index[0]
    dst = edge_index[1]
    neg = jnp.full((8 * N,), NEG, jnp.float32)
    halves = (
        (dst[:_EA], src[:_EA], _gather_A, _segmax_A),
        (dst[_EA:], src[_EA:], _gather_B, _segmax_B),
    )

    def layer(h, W1, b1, W2, b2, gamma, beta, pad):
        parts = []
        for dsth, srch, gat, smx in halves:
            xi, xj = gat(h, dsth, srch)
            h2t = _edge_mlp(xi, xj, W1, b1, W2, b2)
            pr = smx(h2t, dsth, neg).reshape(NSUB, 8, 8, N)
            parts.append(jnp.transpose(pr, (0, 3, 1, 2)).reshape(NSUB, N, 64))
        maxed, stats = _combine_stats(jnp.concatenate(parts, axis=0))
        return _bn_apply(maxed, stats, gamma, beta, pad)

    h = layer(x, W1a, b1a, W2a, b2a, g1, be1, True)
    x_node = layer(h, W1b, b1b, W2b, b2b, g2, be2, False)

    cnt, sums, maxs = _pool_acc(x_node, batch.reshape(N, 1))
    z_proj, x_graph = _final(cnt, sums, maxs, Wp1, bp1, gp, bep, Wp2, bp2)
    return (z_proj, x_node, x_graph)
